# baseline (speedup 1.0000x reference)
import functools

import jax
import jax.numpy as jnp
from jax import lax
from jax.experimental import pallas as pl
from jax.experimental.pallas import tpu as pltpu
from jax.experimental.pallas import tpu_sc as plsc

S = 8192
B = 2
D = 1024
R = S * B
L = 16
CR = 16
NSLOT = 4


def _make_sc_kernel():
    info = plsc.get_sparse_core_info()
    nc, ns = info.num_cores, info.num_subcores
    nw = nc * ns
    rows_per_w = R // nw
    n_chunks = rows_per_w // CR

    mesh = plsc.VectorSubcoreMesh(core_axis_name="c", subcore_axis_name="s")

    @functools.partial(
        pl.kernel,
        mesh=mesh,
        compiler_params=pltpu.CompilerParams(use_tc_tiling_on_sc=True),
        out_type=jax.ShapeDtypeStruct((R, D), jnp.float32),
        scratch_types=[
            pltpu.VMEM((NSLOT, CR, D), jnp.float32),
            pltpu.VMEM((NSLOT, CR // 2, D), jnp.float32),
            pltpu.SemaphoreType.DMA((NSLOT,)),
            pltpu.SemaphoreType.DMA((NSLOT,)),
            pltpu.SemaphoreType.DMA((NSLOT,)),
        ],
    )
    def k(x_hbm, emb_hbm, out_hbm, xbuf, ebuf, sx, se, so):
        wid = lax.axis_index("s") * nc + lax.axis_index("c")
        base = wid * rows_per_w

        def x_copy(g, p):
            r0 = pl.multiple_of(base + g * CR, CR)
            return pltpu.make_async_copy(
                x_hbm.at[pl.ds(r0, CR)], xbuf.at[p], sx.at[p])

        def e_copy(g, p):
            e0 = pl.multiple_of((base + g * CR) // 2, CR // 2)
            return pltpu.make_async_copy(
                emb_hbm.at[pl.ds(e0, CR // 2)], ebuf.at[p], se.at[p])

        def o_copy(g, p):
            r0 = pl.multiple_of(base + g * CR, CR)
            return pltpu.make_async_copy(
                xbuf.at[p], out_hbm.at[pl.ds(r0, CR)], so.at[p])

        def compute(p):
            def erow_body(er, c2):
                for j in range(D // L):
                    sl = pl.ds(j * L, L)
                    e = ebuf[p, er, sl]
                    xbuf[p, 2 * er, sl] = xbuf[p, 2 * er, sl] + e
                    xbuf[p, 2 * er + 1, sl] = xbuf[p, 2 * er + 1, sl] + e
                return c2

            lax.fori_loop(0, CR // 2, erow_body, 0)

        x_copy(0, 0).start()
        e_copy(0, 0).start()

        def tbody(t, carry):
            for k_ in range(NSLOT):
                g = t * NSLOT + k_
                p = k_
                pn = (k_ + 1) % NSLOT

                @pl.when(g >= NSLOT - 1)
                def _():
                    o_copy(g - (NSLOT - 1), pn).wait()

                @pl.when(g + 1 < n_chunks)
                def _():
                    x_copy(g + 1, pn).start()
                    e_copy(g + 1, pn).start()

                x_copy(g, p).wait()
                e_copy(g, p).wait()
                compute(p)
                o_copy(g, p).start()
            return carry

        lax.fori_loop(0, n_chunks // NSLOT, tbody, 0)

        for k_ in range(1, NSLOT):
            o_copy(n_chunks - NSLOT + k_, k_).wait()

    return k


_sc_kernel = _make_sc_kernel()


def kernel(x, emb_weight):
    x2 = x.reshape(R, D)
    out = _sc_kernel(x2, emb_weight)
    return out.reshape(S, B, 1, D)


# speedup vs baseline: 1.1254x; 1.1254x over previous
import functools

import jax
import jax.numpy as jnp
from jax import lax
from jax.experimental import pallas as pl
from jax.experimental.pallas import tpu as pltpu

S = 8192
B = 2
D = 1024
R = S * B
BS = 512


def _tc_body(x_ref, e_ref, o_ref):
    xv = x_ref[...].reshape(BS // 2, 2, D)
    ev = e_ref[...]
    o_ref[...] = (xv + ev[:, None, :]).reshape(BS, D)


_tc_add = pl.pallas_call(
    _tc_body,
    grid=(R // BS,),
    in_specs=[
        pl.BlockSpec((BS, D), lambda i: (i, 0)),
        pl.BlockSpec((BS // 2, D), lambda i: (i, 0)),
    ],
    out_specs=pl.BlockSpec((BS, D), lambda i: (i, 0)),
    out_shape=jax.ShapeDtypeStruct((R, D), jnp.float32),
)


def kernel(x, emb_weight):
    x2 = x.reshape(R, D)
    out = _tc_add(x2, emb_weight)
    return out.reshape(S, B, 1, D)


# speedup vs baseline: 1.1582x; 1.0291x over previous
import functools

import jax
import jax.numpy as jnp
from jax import lax
from jax.experimental import pallas as pl
from jax.experimental.pallas import tpu as pltpu

S = 8192
B = 2
D = 1024
R = S * B
BS = 512

def _tc_body(x_ref, e_ref, o_ref):
    idx0 = lax.broadcasted_iota(jnp.int32, (8, D), 0) // 2
    idx1 = idx0 + 4
    for k in range(BS // 16):
        e8 = e_ref[pl.ds(8 * k, 8), :]
        d0 = jnp.take_along_axis(e8, idx0, axis=0)
        d1 = jnp.take_along_axis(e8, idx1, axis=0)
        o_ref[pl.ds(16 * k, 8), :] = x_ref[pl.ds(16 * k, 8), :] + d0
        o_ref[pl.ds(16 * k + 8, 8), :] = x_ref[pl.ds(16 * k + 8, 8), :] + d1


_tc_add = pl.pallas_call(
    _tc_body,
    grid=(R // BS,),
    in_specs=[
        pl.BlockSpec((BS, D), lambda i: (i, 0)),
        pl.BlockSpec((BS // 2, D), lambda i: (i, 0)),
    ],
    out_specs=pl.BlockSpec((BS, D), lambda i: (i, 0)),
    out_shape=jax.ShapeDtypeStruct((R, D), jnp.float32),
)


def kernel(x, emb_weight):
    x2 = x.reshape(R, D)
    out = _tc_add(x2, emb_weight)
    return out.reshape(S, B, 1, D)


# speedup vs baseline: 5.7194x; 4.9380x over previous
import functools

import jax
import jax.numpy as jnp
from jax import lax
from jax.experimental import pallas as pl
from jax.experimental.pallas import tpu as pltpu
from jax.experimental.pallas import tpu_sc as plsc

S = 8192
B = 2
D = 1024
L = 16
CS = 8
NSLOT = 4


def _make_sc_kernel():
    info = plsc.get_sparse_core_info()
    nc, ns = info.num_cores, info.num_subcores
    nw = nc * ns
    rows_per_w = S // nw
    n_chunks = rows_per_w // CS

    mesh = plsc.VectorSubcoreMesh(core_axis_name="c", subcore_axis_name="s")

    @functools.partial(
        pl.kernel,
        mesh=mesh,
        out_type=jax.ShapeDtypeStruct((S, B, 1, D), jnp.float32),
        scratch_types=[
            pltpu.VMEM((NSLOT, CS, B, 1, D), jnp.float32),
            pltpu.VMEM((NSLOT, CS, D), jnp.float32),
            pltpu.SemaphoreType.DMA((NSLOT,)),
            pltpu.SemaphoreType.DMA((NSLOT,)),
            pltpu.SemaphoreType.DMA((NSLOT,)),
        ],
    )
    def k(x_hbm, emb_hbm, out_hbm, xbuf, ebuf, sx, se, so):
        wid = lax.axis_index("s") * nc + lax.axis_index("c")
        base = wid * rows_per_w

        def x_copy(g, p):
            s0 = base + g * CS
            return pltpu.make_async_copy(
                x_hbm.at[pl.ds(s0, CS)], xbuf.at[p], sx.at[p])

        def e_copy(g, p):
            s0 = base + g * CS
            return pltpu.make_async_copy(
                emb_hbm.at[pl.ds(s0, CS)], ebuf.at[p], se.at[p])

        def o_copy(g, p):
            s0 = base + g * CS
            return pltpu.make_async_copy(
                xbuf.at[p], out_hbm.at[pl.ds(s0, CS)], so.at[p])

        def compute(p):
            def erow_body(er, c2):
                for j in range(D // L):
                    sl = pl.ds(j * L, L)
                    e = ebuf[p, er, sl]
                    xbuf[p, er, 0, 0, sl] = xbuf[p, er, 0, 0, sl] + e
                    xbuf[p, er, 1, 0, sl] = xbuf[p, er, 1, 0, sl] + e
                return c2

            lax.fori_loop(0, CS, erow_body, 0)

        x_copy(0, 0).start()
        e_copy(0, 0).start()

        def tbody(t, carry):
            for k_ in range(NSLOT):
                g = t * NSLOT + k_
                p = k_
                pn = (k_ + 1) % NSLOT

                @pl.when(g >= NSLOT - 1)
                def _():
                    o_copy(g - (NSLOT - 1), pn).wait()

                @pl.when(g + 1 < n_chunks)
                def _():
                    x_copy(g + 1, pn).start()
                    e_copy(g + 1, pn).start()

                x_copy(g, p).wait()
                e_copy(g, p).wait()
                compute(p)
                o_copy(g, p).start()
            return carry

        lax.fori_loop(0, n_chunks // NSLOT, tbody, 0)

        for k_ in range(1, NSLOT):
            o_copy(n_chunks - NSLOT + k_, k_).wait()

    return k


_sc_kernel = _make_sc_kernel()


def kernel(x, emb_weight):
    return _sc_kernel(x, emb_weight)
